# baseline (speedup 1.0000x reference)
import numpy as np
import jax
import jax.numpy as jnp
from jax import lax
from jax.experimental import pallas as pl
from jax.experimental.pallas import tpu as pltpu
from jax.experimental.pallas import tpu_sc as plsc

F32 = jnp.float32
HI = lax.Precision.HIGHEST
KNB = 16
DM = 512
XP = 16


def _dot(a, b):
    return lax.dot_general(a, b, (((1,), (0,)), ((), ())),
                           precision=lax.Precision.DEFAULT,
                           preferred_element_type=F32)


def _dot_t(a, b):
    return lax.dot_general(a, b, (((1,), (1,)), ((), ())), precision=HI,
                           preferred_element_type=F32)


def _pad_cols(a, width):
    return jnp.pad(a, [(0, 0)] * (a.ndim - 1) + [(0, width - a.shape[-1])])


def _pad_rows(a, height):
    return jnp.pad(a, [(0, height - a.shape[0])] + [(0, 0)] * (a.ndim - 1))


def _mlp0(xp, p1, p2):
    B, N, _ = xp.shape
    w1 = _pad_rows(p1["W"], XP)

    def body(x_ref, w1_ref, b1_ref, w2_ref, b2_ref, o_ref):
        h = jnp.maximum(_dot(x_ref[0], w1_ref[...]) + b1_ref[...], 0.0)
        o_ref[0] = _dot(h, w2_ref[...]) + b2_ref[...]

    return pl.pallas_call(
        body,
        grid=(B,),
        in_specs=[
            pl.BlockSpec((1, N, XP), lambda b: (b, 0, 0)),
            pl.BlockSpec(w1.shape, lambda b: (0, 0)),
            pl.BlockSpec((1, 8), lambda b: (0, 0)),
            pl.BlockSpec((8, 8), lambda b: (0, 0)),
            pl.BlockSpec((1, 8), lambda b: (0, 0)),
        ],
        out_specs=pl.BlockSpec((1, N, 8), lambda b: (b, 0, 0)),
        out_shape=jax.ShapeDtypeStruct((B, N, 8), F32),
    )(xp, w1, p1["b"].reshape(1, 8), p2["W"], p2["b"].reshape(1, 8))


def _proj(feats, p):
    B, N, dp = feats.shape
    TN = min(N, 256)

    def body(f_ref, w1_ref, b1_ref, wq_ref, wk_ref, wv_ref,
             q_ref, k_ref, v_ref):
        x1 = _dot(f_ref[0], w1_ref[...]) + b1_ref[...]
        q_ref[0] = _dot(x1, wq_ref[...])
        k_ref[0] = _dot(x1, wk_ref[...])
        v_ref[0] = _dot(x1, wv_ref[...])

    w = pl.BlockSpec((DM, DM), lambda b, m: (0, 0))
    outs = pl.BlockSpec((1, TN, DM), lambda b, m: (b, m, 0))
    return pl.pallas_call(
        body,
        grid=(B, N // TN),
        in_specs=[
            pl.BlockSpec((1, TN, dp), lambda b, m: (b, m, 0)),
            pl.BlockSpec((dp, DM), lambda b, m: (0, 0)),
            pl.BlockSpec((1, DM), lambda b, m: (0, 0)),
            w, w, w,
        ],
        out_specs=[outs, outs, outs],
        out_shape=[jax.ShapeDtypeStruct((B, N, DM), F32)] * 3,
    )(feats, p["fc1"]["W"], p["fc1"]["b"].reshape(1, DM),
      p["w_qs"]["W"], p["w_ks"]["W"], p["w_vs"]["W"])


def _knn(src, dst):
    B, M, _ = src.shape
    N = dst.shape[1]
    TM = min(M, 128)
    dstT = jnp.swapaxes(dst, 1, 2)

    def body(s_ref, d_ref, dt_ref, o_ref):
        b = pl.program_id(0)
        s = s_ref[0]
        d = d_ref[0]
        dt = dt_ref[0]
        mm = lax.dot_general(s, d, (((1,), (1,)), ((), ())),
                             precision=lax.Precision.DEFAULT,
                             preferred_element_type=F32)
        dist = -2.0 * mm + jnp.sum(dt * dt, axis=0, keepdims=True)
        iota = lax.broadcasted_iota(jnp.int32, (TM, N), 1)
        t = dist
        cols = []
        for _ in range(KNB):
            mv = jnp.min(t, axis=-1, keepdims=True)
            am = jnp.min(jnp.where(t == mv, iota, N), axis=-1, keepdims=True)
            cols.append(am)
            t = jnp.where(iota == am, jnp.inf, t)
        o_ref[0] = jnp.concatenate(cols, axis=-1) + b * N

    return pl.pallas_call(
        body,
        grid=(B, M // TM),
        in_specs=[
            pl.BlockSpec((1, TM, XP), lambda b, m: (b, m, 0)),
            pl.BlockSpec((1, N, XP), lambda b, m: (b, 0, 0)),
            pl.BlockSpec((1, XP, N), lambda b, m: (b, 0, 0)),
        ],
        out_specs=pl.BlockSpec((1, TM, KNB), lambda b, m: (b, m, 0)),
        out_shape=jax.ShapeDtypeStruct((B, M, KNB), jnp.int32),
    )(src, dst, dstT)


def _sc_gather(tables, idx):
    G = idx.shape[0]
    orig_dims = [int(t.shape[1]) for t in tables]
    tables = [t if t.shape[1] % 128 == 0 else _pad_cols(t, 128)
              for t in tables]
    dims = [int(t.shape[1]) for t in tables]
    try:
        info = plsc.get_sparse_core_info()
        NC, NS = int(info.num_cores), int(info.num_subcores)
    except Exception:
        NC, NS = 2, 16
    NW = NC * NS
    assert G % NW == 0, (G, NW)
    g_per_w = G // NW
    assert g_per_w % 8 == 0, g_per_w
    row_bytes = sum(4 * D for D in dims)
    chunk = g_per_w
    while chunk * row_bytes > 196608 and chunk > 8:
        chunk //= 2
    nchunk = g_per_w // chunk
    nt = len(tables)
    mesh = plsc.VectorSubcoreMesh(core_axis_name="c", subcore_axis_name="s")

    if nchunk == 1:
        scratch = ([pltpu.VMEM((chunk,), jnp.int32)]
                   + [pltpu.VMEM((chunk, D), F32) for D in dims]
                   + [pltpu.SemaphoreType.DMA])

        def body(*refs):
            t_refs = refs[:nt]
            idx_hbm = refs[nt]
            o_refs = refs[nt + 1: 2 * nt + 1]
            idx_v = refs[2 * nt + 1]
            bufs = refs[2 * nt + 2: 3 * nt + 2]
            sem = refs[3 * nt + 2]
            wid = lax.axis_index("s") * NC + lax.axis_index("c")
            base = wid * g_per_w
            pltpu.sync_copy(idx_hbm.at[pl.ds(base, chunk)], idx_v)
            for t_ref, o_ref, buf in zip(t_refs, o_refs, bufs):
                pltpu.async_copy(t_ref.at[idx_v], buf, sem).wait()
                pltpu.sync_copy(buf, o_ref.at[pl.ds(base, chunk)])
    else:
        scratch = ([pltpu.VMEM((chunk,), jnp.int32)] * 2
                   + [pltpu.VMEM((chunk, D), F32) for D in dims] * 2
                   + [pltpu.SemaphoreType.DMA] * 4)

        def body(*refs):
            t_refs = refs[:nt]
            idx_hbm = refs[nt]
            o_refs = refs[nt + 1: 2 * nt + 1]
            idx_a, idx_b = refs[2 * nt + 1: 2 * nt + 3]
            bufs_a = refs[2 * nt + 3: 3 * nt + 3]
            bufs_b = refs[3 * nt + 3: 4 * nt + 3]
            gs_a, gs_b, os_a, os_b = refs[4 * nt + 3: 4 * nt + 7]
            wid = lax.axis_index("s") * NC + lax.axis_index("c")
            base = wid * g_per_w

            def issue(c, idxv, bufs, gsem):
                pltpu.sync_copy(idx_hbm.at[pl.ds(base + c * chunk, chunk)],
                                idxv)
                for t_ref, buf in zip(t_refs, bufs):
                    pltpu.make_async_copy(t_ref.at[idxv], buf, gsem).start()

            def wait_g(idxv, bufs, gsem):
                for t_ref, buf in zip(t_refs, bufs):
                    pltpu.make_async_copy(t_ref.at[idxv], buf, gsem).wait()

            def put(c, bufs, osem):
                off = base + c * chunk
                for o_ref, buf in zip(o_refs, bufs):
                    pltpu.make_async_copy(buf, o_ref.at[pl.ds(off, chunk)],
                                          osem).start()

            def wait_p(bufs, osem):
                for o_ref, buf in zip(o_refs, bufs):
                    pltpu.make_async_copy(buf, o_ref.at[pl.ds(base, chunk)],
                                          osem).wait()

            issue(0, idx_a, bufs_a, gs_a)

            def loop_body(j, carry):
                c0 = 2 * j
                c1 = c0 + 1

                @pl.when(j > 0)
                def _():
                    wait_p(bufs_b, os_b)
                issue(c1, idx_b, bufs_b, gs_b)
                wait_g(idx_a, bufs_a, gs_a)
                put(c0, bufs_a, os_a)

                @pl.when(c0 + 2 < nchunk)
                def _():
                    wait_p(bufs_a, os_a)
                    issue(c0 + 2, idx_a, bufs_a, gs_a)
                wait_g(idx_b, bufs_b, gs_b)
                put(c1, bufs_b, os_b)
                return carry

            lax.fori_loop(0, nchunk // 2, loop_body, 0)
            wait_p(bufs_a, os_a)
            wait_p(bufs_b, os_b)

    f = pl.kernel(
        body,
        out_type=tuple(jax.ShapeDtypeStruct((G, D), F32) for D in dims),
        mesh=mesh,
        scratch_types=scratch,
    )
    out = f(*tables, idx)
    out = list(out) if isinstance(out, (tuple, list)) else [out]
    return [o if od == d else o[:, :od]
            for o, od, d in zip(out, orig_dims, dims)]


def _attn(p, xyz_pad, feats, q, kg, vg, xg):
    B, N, dp = feats.shape
    TM = min(N, 64)
    wd1 = _pad_rows(p["fc_delta1"]["W"], XP)
    scale = 1.0 / np.sqrt(DM)

    def body(x_ref, f_ref, q_ref, k_ref, v_ref, xg_ref,
             wd1_ref, bd1_ref, wd2_ref, bd2_ref, wg1_ref, bg1_ref,
             wg2_ref, bg2_ref, w2_ref, b2_ref, o_ref):
        xi = x_ref[0]
        xg3 = xg_ref[0]
        delta = (xi.reshape(TM, 1, XP) - xg3).reshape(TM * KNB, XP)
        pos1 = jnp.maximum(_dot(delta, wd1_ref[...]) + bd1_ref[...], 0.0)
        pos = _dot(pos1, wd2_ref[...]) + bd2_ref[...]
        q3 = q_ref[0].reshape(TM, 1, DM)
        g0 = (q3 - k_ref[0]).reshape(TM * KNB, DM) + pos
        g1 = jnp.maximum(_dot(g0, wg1_ref[...]) + bg1_ref[...], 0.0)
        logits = (_dot(g1, wg2_ref[...]) + bg2_ref[...]) * scale
        a3 = logits.reshape(TM, KNB, DM)
        m = jnp.max(a3, axis=1, keepdims=True)
        e = jnp.exp(a3 - m)
        w3 = e / jnp.sum(e, axis=1, keepdims=True)
        vpe = v_ref[0] + pos.reshape(TM, KNB, DM)
        out = jnp.sum(w3 * vpe, axis=1)
        o_ref[0] = _dot(out, w2_ref[...]) + b2_ref[...] + f_ref[0]

    wb = lambda shape: pl.BlockSpec(shape, lambda b, m: (0, 0))
    return pl.pallas_call(
        body,
        grid=(B, N // TM),
        in_specs=[
            pl.BlockSpec((1, TM, XP), lambda b, m: (b, m, 0)),
            pl.BlockSpec((1, TM, dp), lambda b, m: (b, m, 0)),
            pl.BlockSpec((1, TM, DM), lambda b, m: (b, m, 0)),
            pl.BlockSpec((1, TM, KNB, DM), lambda b, m: (b, m, 0, 0)),
            pl.BlockSpec((1, TM, KNB, DM), lambda b, m: (b, m, 0, 0)),
            pl.BlockSpec((1, TM, KNB, XP), lambda b, m: (b, m, 0, 0)),
            wb(wd1.shape), wb((1, DM)),
            wb((DM, DM)), wb((1, DM)),
            wb((DM, DM)), wb((1, DM)),
            wb((DM, DM)), wb((1, DM)),
            wb((DM, dp)), wb((1, dp)),
        ],
        out_specs=pl.BlockSpec((1, TM, dp), lambda b, m: (b, m, 0)),
        out_shape=jax.ShapeDtypeStruct((B, N, dp), F32),
    )(xyz_pad, feats, q, kg, vg, xg,
      wd1, p["fc_delta1"]["b"].reshape(1, DM),
      p["fc_delta2"]["W"], p["fc_delta2"]["b"].reshape(1, DM),
      p["fc_gamma1"]["W"], p["fc_gamma1"]["b"].reshape(1, DM),
      p["fc_gamma2"]["W"], p["fc_gamma2"]["b"].reshape(1, DM),
      p["fc2"]["W"], p["fc2"]["b"].reshape(1, dp))


def _fps(xyzT, npnt):
    B, _, N = xyzT.shape

    def body(xv_ref, xs_ref, o_ref, dist_ref, fsm_ref):
        for b in range(B):
            fsm_ref[b] = 0
        dist_ref[...] = jnp.full((B, 1, N), 1e10, F32)
        iota = lax.broadcasted_iota(jnp.int32, (1, N), 1)

        def step(i, carry):
            for b in range(B):
                f = fsm_ref[b]
                o_ref[b, i] = f + b * N
                cx = xs_ref[b, 0, f]
                cy = xs_ref[b, 1, f]
                cz = xs_ref[b, 2, f]
                d = ((xv_ref[b, 0:1, :] - cx) ** 2
                     + (xv_ref[b, 1:2, :] - cy) ** 2
                     + (xv_ref[b, 2:3, :] - cz) ** 2)
                nd = jnp.minimum(dist_ref[b], d)
                dist_ref[b] = nd
                mv = jnp.max(nd)
                am = jnp.min(jnp.where(nd == mv, iota, N))
                fsm_ref[b] = am.astype(jnp.int32)
            return carry

        lax.fori_loop(0, npnt, step, 0)

    return pl.pallas_call(
        body,
        grid=(1,),
        in_specs=[pl.BlockSpec((B, 3, N), lambda i: (0, 0, 0)),
                  pl.BlockSpec(memory_space=pltpu.SMEM)],
        out_specs=pl.BlockSpec(memory_space=pltpu.SMEM),
        out_shape=jax.ShapeDtypeStruct((B, npnt), jnp.int32),
        scratch_shapes=[pltpu.VMEM((B, 1, N), F32),
                        pltpu.SMEM((B,), jnp.int32)],
    )(xyzT, xyzT)


def _acc_stats(h, s_ref, ss_ref):
    s = jnp.sum(h, axis=0, keepdims=True)
    ss = jnp.sum(h * h, axis=0, keepdims=True)

    @pl.when(pl.program_id(0) == 0)
    def _():
        s_ref[...] = s
        ss_ref[...] = ss

    @pl.when(pl.program_id(0) > 0)
    def _():
        s_ref[...] += s
        ss_ref[...] += ss


def _bn_from_stats(h, s_ref, ss_ref, g_ref, be_ref, n):
    mean = s_ref[...] * (1.0 / n)
    var = ss_ref[...] * (1.0 / n) - mean * mean
    hn = (h - mean) / jnp.sqrt(var + 1e-5)
    return jnp.maximum(g_ref[...] * hn + be_ref[...], 0.0)


def _sa(layers, gsa, nx):
    R = gsa.shape[0]
    M = R // KNB
    C = layers[0]["W"].shape[1]
    w1x = _pad_rows(layers[0]["W"][:3], XP)
    w1p = _pad_rows(jnp.pad(layers[0]["W"][3:], ((XP, 0), (0, 0))), 128)
    TR = min(R, 4096)
    grid = (R // TR,)
    row = lambda width: pl.BlockSpec((TR, width), lambda i: (i, 0))
    cst = lambda shape: pl.BlockSpec(shape, lambda i: (0,) * len(shape))
    stat_spec = [cst((1, C)), cst((1, C))]
    stat_shape = [jax.ShapeDtypeStruct((1, C), F32)] * 2

    def k1(g_ref, nx_ref, w1x_ref, w1p_ref, b1_ref,
           h_ref, s_ref, ss_ref):
        g = g_ref[...]
        gx3 = g[:, :XP].reshape(TR // KNB, KNB, XP)
        nx3 = nx_ref[...].reshape(TR // KNB, 1, XP)
        gnorm = (gx3 - nx3).reshape(TR, XP)
        h = (_dot(gnorm, w1x_ref[...]) + _dot(g, w1p_ref[...])
             + b1_ref[...])
        h_ref[...] = h
        _acc_stats(h, s_ref, ss_ref)

    h1, s1, ss1 = pl.pallas_call(
        k1, grid=grid,
        in_specs=[row(128),
                  pl.BlockSpec((TR // KNB, XP), lambda i: (i, 0)),
                  cst(w1x.shape), cst(w1p.shape), cst((1, C))],
        out_specs=[row(C)] + stat_spec,
        out_shape=[jax.ShapeDtypeStruct((R, C), F32)] + stat_shape,
    )(gsa, nx, w1x, w1p, layers[0]["b"].reshape(1, C))

    def k2(h_ref, s1_ref, ss1_ref, g1_ref, be1_ref, w2_ref, b2_ref,
           h2_ref, s_ref, ss_ref):
        hn = _bn_from_stats(h_ref[...], s1_ref, ss1_ref, g1_ref, be1_ref, R)
        h2 = _dot(hn, w2_ref[...]) + b2_ref[...]
        h2_ref[...] = h2
        _acc_stats(h2, s_ref, ss_ref)

    h2, s2, ss2 = pl.pallas_call(
        k2, grid=grid,
        in_specs=[row(C), cst((1, C)), cst((1, C)), cst((1, C)), cst((1, C)),
                  cst((C, C)), cst((1, C))],
        out_specs=[row(C)] + stat_spec,
        out_shape=[jax.ShapeDtypeStruct((R, C), F32)] + stat_shape,
    )(h1, s1, ss1, layers[0]["gamma"].reshape(1, C),
      layers[0]["beta"].reshape(1, C), layers[1]["W"],
      layers[1]["b"].reshape(1, C))

    def k3(h2_ref, s2_ref, ss2_ref, g2_ref, be2_ref, o_ref):
        hn = _bn_from_stats(h2_ref[...], s2_ref, ss2_ref, g2_ref, be2_ref, R)
        o_ref[...] = jnp.max(hn.reshape(TR // KNB, KNB, C), axis=1)

    return pl.pallas_call(
        k3, grid=grid,
        in_specs=[row(C), cst((1, C)), cst((1, C)), cst((1, C)),
                  cst((1, C))],
        out_specs=pl.BlockSpec((TR // KNB, C), lambda i: (i, 0)),
        out_shape=jax.ShapeDtypeStruct((M, C), F32),
    )(h2, s2, ss2, layers[1]["gamma"].reshape(1, C),
      layers[1]["beta"].reshape(1, C))


def _tblock(p, xyz_pad, feats):
    B, N, dp = feats.shape
    q, kp, vp = _proj(feats, p)
    knn_g = _knn(xyz_pad, xyz_pad)
    gflat = knn_g.reshape(B * N * KNB)
    kg, vg, xg = _sc_gather(
        [kp.reshape(B * N, DM), vp.reshape(B * N, DM),
         xyz_pad.reshape(B * N, XP)], gflat)
    return _attn(p, xyz_pad, feats, q,
                 kg.reshape(B, N, KNB, DM), vg.reshape(B, N, KNB, DM),
                 xg.reshape(B, N, KNB, XP))


def kernel(x, params):
    B, N, _ = x.shape
    xyz = x[..., :3]
    xyz_pad = _pad_cols(xyz, XP)
    feats = _mlp0(_pad_cols(x, XP), params["fc1a"], params["fc1b"])
    points = _tblock(params["t1"], xyz_pad, feats)
    xyz_and_feats = [(xyz, points)]
    cur_xyzp, cur_points = xyz_pad, points
    cur_n = N
    for i in range(4):
        npnt = N // 2 ** (i + 1)
        fps_g = _fps(jnp.swapaxes(cur_xyzp[..., :3], 1, 2), npnt)
        fps_flat = fps_g.reshape(B * npnt)
        (new_xyzp_flat,) = _sc_gather(
            [cur_xyzp.reshape(B * cur_n, XP)], fps_flat)
        new_xyzp = new_xyzp_flat.reshape(B, npnt, XP)
        knn_g = _knn(new_xyzp, cur_xyzp)
        gflat = knn_g.reshape(B * npnt * KNB)
        sa_tab = _pad_cols(jnp.concatenate([cur_xyzp, cur_points], axis=-1),
                           128)
        (gsa,) = _sc_gather([sa_tab.reshape(B * cur_n, 128)], gflat)
        new_points = _sa(params["td"][i], gsa,
                         new_xyzp.reshape(B * npnt, XP))
        new_points = new_points.reshape(B, npnt, -1)
        cur_points = _tblock(params["tf"][i], new_xyzp, new_points)
        cur_xyzp = new_xyzp
        cur_n = npnt
        xyz_and_feats.append((new_xyzp[..., :3], cur_points))
    return cur_points, xyz_and_feats


# speedup vs baseline: 1.3347x; 1.3347x over previous
import numpy as np
import jax
import jax.numpy as jnp
from jax import lax
from jax.experimental import pallas as pl
from jax.experimental.pallas import tpu as pltpu
from jax.experimental.pallas import tpu_sc as plsc

F32 = jnp.float32
HI = lax.Precision.HIGHEST
KNB = 16
DM = 512
XP = 16


def _dot(a, b):
    return lax.dot_general(a, b, (((1,), (0,)), ((), ())),
                           precision=lax.Precision.DEFAULT,
                           preferred_element_type=F32)


def _dot_t(a, b):
    return lax.dot_general(a, b, (((1,), (1,)), ((), ())), precision=HI,
                           preferred_element_type=F32)


def _pad_cols(a, width):
    return jnp.pad(a, [(0, 0)] * (a.ndim - 1) + [(0, width - a.shape[-1])])


def _pad_rows(a, height):
    return jnp.pad(a, [(0, height - a.shape[0])] + [(0, 0)] * (a.ndim - 1))


def _mlp0(xp, p1, p2):
    B, N, _ = xp.shape
    w1 = _pad_rows(p1["W"], XP)

    def body(x_ref, w1_ref, b1_ref, w2_ref, b2_ref, o_ref):
        h = jnp.maximum(_dot(x_ref[0], w1_ref[...]) + b1_ref[...], 0.0)
        o_ref[0] = _dot(h, w2_ref[...]) + b2_ref[...]

    return pl.pallas_call(
        body,
        grid=(B,),
        in_specs=[
            pl.BlockSpec((1, N, XP), lambda b: (b, 0, 0)),
            pl.BlockSpec(w1.shape, lambda b: (0, 0)),
            pl.BlockSpec((1, 8), lambda b: (0, 0)),
            pl.BlockSpec((8, 8), lambda b: (0, 0)),
            pl.BlockSpec((1, 8), lambda b: (0, 0)),
        ],
        out_specs=pl.BlockSpec((1, N, 8), lambda b: (b, 0, 0)),
        out_shape=jax.ShapeDtypeStruct((B, N, 8), F32),
    )(xp, w1, p1["b"].reshape(1, 8), p2["W"], p2["b"].reshape(1, 8))


def _proj(feats, p):
    B, N, dp = feats.shape
    TN = min(N, 256)

    def body(f_ref, w1_ref, b1_ref, wq_ref, wk_ref, wv_ref,
             q_ref, k_ref, v_ref):
        x1 = _dot(f_ref[0], w1_ref[...]) + b1_ref[...]
        q_ref[0] = _dot(x1, wq_ref[...])
        k_ref[0] = _dot(x1, wk_ref[...])
        v_ref[0] = _dot(x1, wv_ref[...])

    w = pl.BlockSpec((DM, DM), lambda b, m: (0, 0))
    outs = pl.BlockSpec((1, TN, DM), lambda b, m: (b, m, 0))
    return pl.pallas_call(
        body,
        grid=(B, N // TN),
        in_specs=[
            pl.BlockSpec((1, TN, dp), lambda b, m: (b, m, 0)),
            pl.BlockSpec((dp, DM), lambda b, m: (0, 0)),
            pl.BlockSpec((1, DM), lambda b, m: (0, 0)),
            w, w, w,
        ],
        out_specs=[outs, outs, outs],
        out_shape=[jax.ShapeDtypeStruct((B, N, DM), F32)] * 3,
    )(feats, p["fc1"]["W"], p["fc1"]["b"].reshape(1, DM),
      p["w_qs"]["W"], p["w_ks"]["W"], p["w_vs"]["W"])


def _knn(src, dst):
    B, M, _ = src.shape
    N = dst.shape[1]
    TM = min(M, 128)
    dstT = jnp.swapaxes(dst, 1, 2)

    def body(s_ref, d_ref, dt_ref, o_ref):
        b = pl.program_id(0)
        s = s_ref[0]
        d = d_ref[0]
        dt = dt_ref[0]
        mm = lax.dot_general(s, d, (((1,), (1,)), ((), ())),
                             precision=lax.Precision.DEFAULT,
                             preferred_element_type=F32)
        dist = -2.0 * mm + jnp.sum(dt * dt, axis=0, keepdims=True)
        iota = lax.broadcasted_iota(jnp.int32, (TM, N), 1)
        t = dist
        cols = []
        for _ in range(KNB):
            mv = jnp.min(t, axis=-1, keepdims=True)
            am = jnp.min(jnp.where(t == mv, iota, N), axis=-1, keepdims=True)
            cols.append(am)
            t = jnp.where(iota == am, jnp.inf, t)
        o_ref[0] = jnp.concatenate(cols, axis=-1) + b * N

    return pl.pallas_call(
        body,
        grid=(B, M // TM),
        in_specs=[
            pl.BlockSpec((1, TM, XP), lambda b, m: (b, m, 0)),
            pl.BlockSpec((1, N, XP), lambda b, m: (b, 0, 0)),
            pl.BlockSpec((1, XP, N), lambda b, m: (b, 0, 0)),
        ],
        out_specs=pl.BlockSpec((1, TM, KNB), lambda b, m: (b, m, 0)),
        out_shape=jax.ShapeDtypeStruct((B, M, KNB), jnp.int32),
    )(src, dst, dstT)


def _sc_gather(tables, idx):
    G = idx.shape[0]
    orig_dims = [int(t.shape[1]) for t in tables]
    tables = [t if t.shape[1] % 128 == 0 else _pad_cols(t, 128)
              for t in tables]
    dims = [int(t.shape[1]) for t in tables]
    try:
        info = plsc.get_sparse_core_info()
        NC, NS = int(info.num_cores), int(info.num_subcores)
    except Exception:
        NC, NS = 2, 16
    NW = NC * NS
    assert G % NW == 0, (G, NW)
    g_per_w = G // NW
    assert g_per_w % 8 == 0, g_per_w
    row_bytes = sum(4 * D for D in dims)
    chunk = g_per_w
    while chunk * row_bytes > 196608 and chunk > 8:
        chunk //= 2
    nchunk = g_per_w // chunk
    nt = len(tables)
    mesh = plsc.VectorSubcoreMesh(core_axis_name="c", subcore_axis_name="s")

    if nchunk == 1:
        scratch = ([pltpu.VMEM((chunk,), jnp.int32)]
                   + [pltpu.VMEM((chunk, D), F32) for D in dims]
                   + [pltpu.SemaphoreType.DMA])

        def body(*refs):
            t_refs = refs[:nt]
            idx_hbm = refs[nt]
            o_refs = refs[nt + 1: 2 * nt + 1]
            idx_v = refs[2 * nt + 1]
            bufs = refs[2 * nt + 2: 3 * nt + 2]
            sem = refs[3 * nt + 2]
            wid = lax.axis_index("s") * NC + lax.axis_index("c")
            base = wid * g_per_w
            pltpu.sync_copy(idx_hbm.at[pl.ds(base, chunk)], idx_v)
            for t_ref, o_ref, buf in zip(t_refs, o_refs, bufs):
                pltpu.async_copy(t_ref.at[idx_v], buf, sem).wait()
                pltpu.sync_copy(buf, o_ref.at[pl.ds(base, chunk)])
    else:
        scratch = ([pltpu.VMEM((chunk,), jnp.int32)] * 2
                   + [pltpu.VMEM((chunk, D), F32) for D in dims] * 2
                   + [pltpu.SemaphoreType.DMA] * 4)

        def body(*refs):
            t_refs = refs[:nt]
            idx_hbm = refs[nt]
            o_refs = refs[nt + 1: 2 * nt + 1]
            idx_a, idx_b = refs[2 * nt + 1: 2 * nt + 3]
            bufs_a = refs[2 * nt + 3: 3 * nt + 3]
            bufs_b = refs[3 * nt + 3: 4 * nt + 3]
            gs_a, gs_b, os_a, os_b = refs[4 * nt + 3: 4 * nt + 7]
            wid = lax.axis_index("s") * NC + lax.axis_index("c")
            base = wid * g_per_w

            def issue(c, idxv, bufs, gsem):
                pltpu.sync_copy(idx_hbm.at[pl.ds(base + c * chunk, chunk)],
                                idxv)
                for t_ref, buf in zip(t_refs, bufs):
                    pltpu.make_async_copy(t_ref.at[idxv], buf, gsem).start()

            def wait_g(idxv, bufs, gsem):
                for t_ref, buf in zip(t_refs, bufs):
                    pltpu.make_async_copy(t_ref.at[idxv], buf, gsem).wait()

            def put(c, bufs, osem):
                off = base + c * chunk
                for o_ref, buf in zip(o_refs, bufs):
                    pltpu.make_async_copy(buf, o_ref.at[pl.ds(off, chunk)],
                                          osem).start()

            def wait_p(bufs, osem):
                for o_ref, buf in zip(o_refs, bufs):
                    pltpu.make_async_copy(buf, o_ref.at[pl.ds(base, chunk)],
                                          osem).wait()

            issue(0, idx_a, bufs_a, gs_a)

            def loop_body(j, carry):
                c0 = 2 * j
                c1 = c0 + 1

                @pl.when(j > 0)
                def _():
                    wait_p(bufs_b, os_b)
                issue(c1, idx_b, bufs_b, gs_b)
                wait_g(idx_a, bufs_a, gs_a)
                put(c0, bufs_a, os_a)

                @pl.when(c0 + 2 < nchunk)
                def _():
                    wait_p(bufs_a, os_a)
                    issue(c0 + 2, idx_a, bufs_a, gs_a)
                wait_g(idx_b, bufs_b, gs_b)
                put(c1, bufs_b, os_b)
                return carry

            lax.fori_loop(0, nchunk // 2, loop_body, 0)
            wait_p(bufs_a, os_a)
            wait_p(bufs_b, os_b)

    f = pl.kernel(
        body,
        out_type=tuple(jax.ShapeDtypeStruct((G, D), F32) for D in dims),
        mesh=mesh,
        scratch_types=scratch,
    )
    out = f(*tables, idx)
    out = list(out) if isinstance(out, (tuple, list)) else [out]
    return [o if od == d else o[:, :od]
            for o, od, d in zip(out, orig_dims, dims)]


def _attn(p, xyz_pad, feats, q, kg, vg, xg):
    B, N, dp = feats.shape
    TM = min(N, 128)
    wd1 = _pad_rows(p["fc_delta1"]["W"], XP)
    scale = 1.0 / np.sqrt(DM)

    def body(x_ref, f_ref, q_ref, k_ref, v_ref, xg_ref,
             wd1_ref, bd1_ref, wd2_ref, bd2_ref, wg1_ref, bg1_ref,
             wg2_ref, bg2_ref, w2_ref, b2_ref, o_ref):
        xi = x_ref[0]
        xg3 = xg_ref[0]
        delta = (xi.reshape(TM, 1, XP) - xg3).reshape(TM * KNB, XP)
        pos1 = jnp.maximum(_dot(delta, wd1_ref[...]) + bd1_ref[...], 0.0)
        pos = _dot(pos1, wd2_ref[...]) + bd2_ref[...]
        q3 = q_ref[0].reshape(TM, 1, DM)
        g0 = (q3 - k_ref[0]).reshape(TM * KNB, DM) + pos
        g1 = jnp.maximum(_dot(g0, wg1_ref[...]) + bg1_ref[...], 0.0)
        logits = (_dot(g1, wg2_ref[...]) + bg2_ref[...]) * scale
        a3 = logits.reshape(TM, KNB, DM)
        m = jnp.max(a3, axis=1, keepdims=True)
        e = jnp.exp(a3 - m)
        w3 = e / jnp.sum(e, axis=1, keepdims=True)
        vpe = v_ref[0] + pos.reshape(TM, KNB, DM)
        out = jnp.sum(w3 * vpe, axis=1)
        o_ref[0] = _dot(out, w2_ref[...]) + b2_ref[...] + f_ref[0]

    wb = lambda shape: pl.BlockSpec(shape, lambda b, m: (0, 0))
    return pl.pallas_call(
        body,
        grid=(B, N // TM),
        in_specs=[
            pl.BlockSpec((1, TM, XP), lambda b, m: (b, m, 0)),
            pl.BlockSpec((1, TM, dp), lambda b, m: (b, m, 0)),
            pl.BlockSpec((1, TM, DM), lambda b, m: (b, m, 0)),
            pl.BlockSpec((1, TM, KNB, DM), lambda b, m: (b, m, 0, 0)),
            pl.BlockSpec((1, TM, KNB, DM), lambda b, m: (b, m, 0, 0)),
            pl.BlockSpec((1, TM, KNB, XP), lambda b, m: (b, m, 0, 0)),
            wb(wd1.shape), wb((1, DM)),
            wb((DM, DM)), wb((1, DM)),
            wb((DM, DM)), wb((1, DM)),
            wb((DM, DM)), wb((1, DM)),
            wb((DM, dp)), wb((1, dp)),
        ],
        out_specs=pl.BlockSpec((1, TM, dp), lambda b, m: (b, m, 0)),
        out_shape=jax.ShapeDtypeStruct((B, N, dp), F32),
    )(xyz_pad, feats, q, kg, vg, xg,
      wd1, p["fc_delta1"]["b"].reshape(1, DM),
      p["fc_delta2"]["W"], p["fc_delta2"]["b"].reshape(1, DM),
      p["fc_gamma1"]["W"], p["fc_gamma1"]["b"].reshape(1, DM),
      p["fc_gamma2"]["W"], p["fc_gamma2"]["b"].reshape(1, DM),
      p["fc2"]["W"], p["fc2"]["b"].reshape(1, dp))


def _fps(xyzT, npnt):
    B, _, N = xyzT.shape

    def body(x_ref, o_ref):
        xv = x_ref[...]
        iota = lax.broadcasted_iota(jnp.int32, (B, 1, N), 2)
        bofs = lax.broadcasted_iota(jnp.int32, (B, 1, 1), 0) * N

        def step(i, carry):
            dist, f = carry
            o_ref[:, pl.ds(i, 1), :] = f + bofs
            mask = (iota == f).astype(F32)
            cm = jnp.sum(xv * mask, axis=2, keepdims=True)
            d = jnp.sum((xv - cm) ** 2, axis=1, keepdims=True)
            dist = jnp.minimum(dist, d)
            mv = jnp.max(dist, axis=2, keepdims=True)
            f2 = jnp.min(jnp.where(dist == mv, iota, N), axis=2,
                         keepdims=True)
            return dist, f2

        lax.fori_loop(0, npnt, step,
                      (jnp.full((B, 1, N), 1e10, F32),
                       jnp.zeros((B, 1, 1), jnp.int32)))

    return pl.pallas_call(
        body,
        grid=(1,),
        in_specs=[pl.BlockSpec((B, 4, N), lambda i: (0, 0, 0))],
        out_specs=pl.BlockSpec((B, npnt, 1), lambda i: (0, 0, 0)),
        out_shape=jax.ShapeDtypeStruct((B, npnt, 1), jnp.int32),
    )(xyzT)


def _acc_stats(h, s_ref, ss_ref):
    s = jnp.sum(h, axis=0, keepdims=True)
    ss = jnp.sum(h * h, axis=0, keepdims=True)

    @pl.when(pl.program_id(0) == 0)
    def _():
        s_ref[...] = s
        ss_ref[...] = ss

    @pl.when(pl.program_id(0) > 0)
    def _():
        s_ref[...] += s
        ss_ref[...] += ss


def _bn_from_stats(h, s_ref, ss_ref, g_ref, be_ref, n):
    mean = s_ref[...] * (1.0 / n)
    var = ss_ref[...] * (1.0 / n) - mean * mean
    hn = (h - mean) / jnp.sqrt(var + 1e-5)
    return jnp.maximum(g_ref[...] * hn + be_ref[...], 0.0)


def _sa(layers, gsa, nx):
    R = gsa.shape[0]
    M = R // KNB
    C = layers[0]["W"].shape[1]
    w1x = _pad_rows(layers[0]["W"][:3], XP)
    w1p = _pad_rows(jnp.pad(layers[0]["W"][3:], ((XP, 0), (0, 0))), 128)
    TR = min(R, 4096)
    grid = (R // TR,)
    row = lambda width: pl.BlockSpec((TR, width), lambda i: (i, 0))
    cst = lambda shape: pl.BlockSpec(shape, lambda i: (0,) * len(shape))
    stat_spec = [cst((1, C)), cst((1, C))]
    stat_shape = [jax.ShapeDtypeStruct((1, C), F32)] * 2

    def k1(g_ref, nx_ref, w1x_ref, w1p_ref, b1_ref,
           h_ref, s_ref, ss_ref):
        g = g_ref[...]
        gx3 = g[:, :XP].reshape(TR // KNB, KNB, XP)
        nx3 = nx_ref[...].reshape(TR // KNB, 1, XP)
        gnorm = (gx3 - nx3).reshape(TR, XP)
        h = (_dot(gnorm, w1x_ref[...]) + _dot(g, w1p_ref[...])
             + b1_ref[...])
        h_ref[...] = h
        _acc_stats(h, s_ref, ss_ref)

    h1, s1, ss1 = pl.pallas_call(
        k1, grid=grid,
        in_specs=[row(128),
                  pl.BlockSpec((TR // KNB, XP), lambda i: (i, 0)),
                  cst(w1x.shape), cst(w1p.shape), cst((1, C))],
        out_specs=[row(C)] + stat_spec,
        out_shape=[jax.ShapeDtypeStruct((R, C), F32)] + stat_shape,
    )(gsa, nx, w1x, w1p, layers[0]["b"].reshape(1, C))

    def k2(h_ref, s1_ref, ss1_ref, g1_ref, be1_ref, w2_ref, b2_ref,
           h2_ref, s_ref, ss_ref):
        hn = _bn_from_stats(h_ref[...], s1_ref, ss1_ref, g1_ref, be1_ref, R)
        h2 = _dot(hn, w2_ref[...]) + b2_ref[...]
        h2_ref[...] = h2
        _acc_stats(h2, s_ref, ss_ref)

    h2, s2, ss2 = pl.pallas_call(
        k2, grid=grid,
        in_specs=[row(C), cst((1, C)), cst((1, C)), cst((1, C)), cst((1, C)),
                  cst((C, C)), cst((1, C))],
        out_specs=[row(C)] + stat_spec,
        out_shape=[jax.ShapeDtypeStruct((R, C), F32)] + stat_shape,
    )(h1, s1, ss1, layers[0]["gamma"].reshape(1, C),
      layers[0]["beta"].reshape(1, C), layers[1]["W"],
      layers[1]["b"].reshape(1, C))

    def k3(h2_ref, s2_ref, ss2_ref, g2_ref, be2_ref, o_ref):
        hn = _bn_from_stats(h2_ref[...], s2_ref, ss2_ref, g2_ref, be2_ref, R)
        o_ref[...] = jnp.max(hn.reshape(TR // KNB, KNB, C), axis=1)

    return pl.pallas_call(
        k3, grid=grid,
        in_specs=[row(C), cst((1, C)), cst((1, C)), cst((1, C)),
                  cst((1, C))],
        out_specs=pl.BlockSpec((TR // KNB, C), lambda i: (i, 0)),
        out_shape=jax.ShapeDtypeStruct((M, C), F32),
    )(h2, s2, ss2, layers[1]["gamma"].reshape(1, C),
      layers[1]["beta"].reshape(1, C))


def _tblock(p, xyz_pad, feats):
    B, N, dp = feats.shape
    q, kp, vp = _proj(feats, p)
    knn_g = _knn(xyz_pad, xyz_pad)
    gflat = knn_g.reshape(B * N * KNB)
    kg, vg, xg = _sc_gather(
        [kp.reshape(B * N, DM), vp.reshape(B * N, DM),
         xyz_pad.reshape(B * N, XP)], gflat)
    return _attn(p, xyz_pad, feats, q,
                 kg.reshape(B, N, KNB, DM), vg.reshape(B, N, KNB, DM),
                 xg.reshape(B, N, KNB, XP))


def kernel(x, params):
    B, N, _ = x.shape
    xyz = x[..., :3]
    xyz_pad = _pad_cols(xyz, XP)
    feats = _mlp0(_pad_cols(x, XP), params["fc1a"], params["fc1b"])
    points = _tblock(params["t1"], xyz_pad, feats)
    xyz_and_feats = [(xyz, points)]
    cur_xyzp, cur_points = xyz_pad, points
    cur_n = N
    for i in range(4):
        npnt = N // 2 ** (i + 1)
        fps_g = _fps(jnp.swapaxes(cur_xyzp[..., :4], 1, 2), npnt)
        fps_flat = fps_g.reshape(B * npnt)
        (new_xyzp_flat,) = _sc_gather(
            [cur_xyzp.reshape(B * cur_n, XP)], fps_flat)
        new_xyzp = new_xyzp_flat.reshape(B, npnt, XP)
        knn_g = _knn(new_xyzp, cur_xyzp)
        gflat = knn_g.reshape(B * npnt * KNB)
        sa_tab = _pad_cols(jnp.concatenate([cur_xyzp, cur_points], axis=-1),
                           128)
        (gsa,) = _sc_gather([sa_tab.reshape(B * cur_n, 128)], gflat)
        new_points = _sa(params["td"][i], gsa,
                         new_xyzp.reshape(B * npnt, XP))
        new_points = new_points.reshape(B, npnt, -1)
        cur_points = _tblock(params["tf"][i], new_xyzp, new_points)
        cur_xyzp = new_xyzp
        cur_n = npnt
        xyz_and_feats.append((new_xyzp[..., :3], cur_points))
    return cur_points, xyz_and_feats


# speedup vs baseline: 1.3367x; 1.0015x over previous
import numpy as np
import jax
import jax.numpy as jnp
from jax import lax
from jax.experimental import pallas as pl
from jax.experimental.pallas import tpu as pltpu
from jax.experimental.pallas import tpu_sc as plsc

F32 = jnp.float32
HI = lax.Precision.HIGHEST
KNB = 16
DM = 512
XP = 16


def _dot(a, b):
    return lax.dot_general(a, b, (((1,), (0,)), ((), ())),
                           precision=lax.Precision.DEFAULT,
                           preferred_element_type=F32)


def _dot_t(a, b):
    return lax.dot_general(a, b, (((1,), (1,)), ((), ())), precision=HI,
                           preferred_element_type=F32)


def _pad_cols(a, width):
    return jnp.pad(a, [(0, 0)] * (a.ndim - 1) + [(0, width - a.shape[-1])])


def _pad_rows(a, height):
    return jnp.pad(a, [(0, height - a.shape[0])] + [(0, 0)] * (a.ndim - 1))


def _mlp0(xp, p1, p2):
    B, N, _ = xp.shape
    w1 = _pad_rows(p1["W"], XP)

    def body(x_ref, w1_ref, b1_ref, w2_ref, b2_ref, o_ref):
        h = jnp.maximum(_dot(x_ref[0], w1_ref[...]) + b1_ref[...], 0.0)
        o_ref[0] = _dot(h, w2_ref[...]) + b2_ref[...]

    return pl.pallas_call(
        body,
        grid=(B,),
        in_specs=[
            pl.BlockSpec((1, N, XP), lambda b: (b, 0, 0)),
            pl.BlockSpec(w1.shape, lambda b: (0, 0)),
            pl.BlockSpec((1, 8), lambda b: (0, 0)),
            pl.BlockSpec((8, 8), lambda b: (0, 0)),
            pl.BlockSpec((1, 8), lambda b: (0, 0)),
        ],
        out_specs=pl.BlockSpec((1, N, 8), lambda b: (b, 0, 0)),
        out_shape=jax.ShapeDtypeStruct((B, N, 8), F32),
    )(xp, w1, p1["b"].reshape(1, 8), p2["W"], p2["b"].reshape(1, 8))


def _proj(feats, p):
    B, N, dp = feats.shape
    TN = min(N, 256)

    def body(f_ref, w1_ref, b1_ref, wq_ref, wk_ref, wv_ref,
             q_ref, k_ref, v_ref):
        x1 = _dot(f_ref[0], w1_ref[...]) + b1_ref[...]
        q_ref[0] = _dot(x1, wq_ref[...])
        k_ref[0] = _dot(x1, wk_ref[...])
        v_ref[0] = _dot(x1, wv_ref[...])

    w = pl.BlockSpec((DM, DM), lambda b, m: (0, 0))
    outs = pl.BlockSpec((1, TN, DM), lambda b, m: (b, m, 0))
    return pl.pallas_call(
        body,
        grid=(B, N // TN),
        in_specs=[
            pl.BlockSpec((1, TN, dp), lambda b, m: (b, m, 0)),
            pl.BlockSpec((dp, DM), lambda b, m: (0, 0)),
            pl.BlockSpec((1, DM), lambda b, m: (0, 0)),
            w, w, w,
        ],
        out_specs=[outs, outs, outs],
        out_shape=[jax.ShapeDtypeStruct((B, N, DM), F32)] * 3,
    )(feats, p["fc1"]["W"], p["fc1"]["b"].reshape(1, DM),
      p["w_qs"]["W"], p["w_ks"]["W"], p["w_vs"]["W"])


def _knn(src, dst):
    B, M, _ = src.shape
    N = dst.shape[1]
    TM = min(M, 128)
    dstT = jnp.swapaxes(dst, 1, 2)

    def body(s_ref, d_ref, dt_ref, o_ref):
        b = pl.program_id(0)
        s = s_ref[0]
        d = d_ref[0]
        dt = dt_ref[0]
        mm = lax.dot_general(s, d, (((1,), (1,)), ((), ())),
                             precision=lax.Precision.DEFAULT,
                             preferred_element_type=F32)
        dist = -2.0 * mm + jnp.sum(dt * dt, axis=0, keepdims=True)
        iota = lax.broadcasted_iota(jnp.int32, (TM, N), 1)
        t = dist
        cols = []
        for _ in range(KNB):
            mv = jnp.min(t, axis=-1, keepdims=True)
            am = jnp.min(jnp.where(t == mv, iota, N), axis=-1, keepdims=True)
            cols.append(am)
            t = jnp.where(iota == am, jnp.inf, t)
        o_ref[0] = jnp.concatenate(cols, axis=-1) + b * N

    return pl.pallas_call(
        body,
        grid=(B, M // TM),
        in_specs=[
            pl.BlockSpec((1, TM, XP), lambda b, m: (b, m, 0)),
            pl.BlockSpec((1, N, XP), lambda b, m: (b, 0, 0)),
            pl.BlockSpec((1, XP, N), lambda b, m: (b, 0, 0)),
        ],
        out_specs=pl.BlockSpec((1, TM, KNB), lambda b, m: (b, m, 0)),
        out_shape=jax.ShapeDtypeStruct((B, M, KNB), jnp.int32),
    )(src, dst, dstT)


def _sc_gather(tables, idx):
    G = idx.shape[0]
    orig_dims = []
    padded = []
    for t in tables:
        if t.ndim == 2 and t.shape[1] % 128 != 0:
            orig_dims.append(int(t.shape[1]))
            padded.append(_pad_cols(t, 128))
        else:
            orig_dims.append(None)
            padded.append(t)
    tables = padded
    try:
        info = plsc.get_sparse_core_info()
        NC, NS = int(info.num_cores), int(info.num_subcores)
    except Exception:
        NC, NS = 2, 16
    NW = NC * NS
    assert G % NW == 0, (G, NW)
    g_per_w = G // NW
    assert g_per_w % 8 == 0, g_per_w
    row_shapes = [tuple(t.shape[1:]) for t in tables]
    row_dtypes = [t.dtype for t in tables]
    row_bytes = sum(int(np.prod(s)) * np.dtype(d).itemsize
                    for s, d in zip(row_shapes, row_dtypes))
    chunk = g_per_w
    while chunk * row_bytes > 196608 and chunk > 8:
        chunk //= 2
    nchunk = g_per_w // chunk
    nt = len(tables)
    mesh = plsc.VectorSubcoreMesh(core_axis_name="c", subcore_axis_name="s")

    bufs_t = [pltpu.VMEM((chunk,) + s, d)
              for s, d in zip(row_shapes, row_dtypes)]
    if nchunk == 1:
        scratch = ([pltpu.VMEM((chunk,), jnp.int32)]
                   + bufs_t
                   + [pltpu.SemaphoreType.DMA])

        def body(*refs):
            t_refs = refs[:nt]
            idx_hbm = refs[nt]
            o_refs = refs[nt + 1: 2 * nt + 1]
            idx_v = refs[2 * nt + 1]
            bufs = refs[2 * nt + 2: 3 * nt + 2]
            sem = refs[3 * nt + 2]
            wid = lax.axis_index("s") * NC + lax.axis_index("c")
            base = wid * g_per_w
            pltpu.sync_copy(idx_hbm.at[pl.ds(base, chunk)], idx_v)
            for t_ref, o_ref, buf in zip(t_refs, o_refs, bufs):
                pltpu.async_copy(t_ref.at[idx_v], buf, sem).wait()
                pltpu.sync_copy(buf, o_ref.at[pl.ds(base, chunk)])
    else:
        scratch = ([pltpu.VMEM((chunk,), jnp.int32)] * 2
                   + bufs_t * 2
                   + [pltpu.SemaphoreType.DMA] * 4)

        def body(*refs):
            t_refs = refs[:nt]
            idx_hbm = refs[nt]
            o_refs = refs[nt + 1: 2 * nt + 1]
            idx_a, idx_b = refs[2 * nt + 1: 2 * nt + 3]
            bufs_a = refs[2 * nt + 3: 3 * nt + 3]
            bufs_b = refs[3 * nt + 3: 4 * nt + 3]
            gs_a, gs_b, os_a, os_b = refs[4 * nt + 3: 4 * nt + 7]
            wid = lax.axis_index("s") * NC + lax.axis_index("c")
            base = wid * g_per_w

            def issue(c, idxv, bufs, gsem):
                pltpu.sync_copy(idx_hbm.at[pl.ds(base + c * chunk, chunk)],
                                idxv)
                for t_ref, buf in zip(t_refs, bufs):
                    pltpu.make_async_copy(t_ref.at[idxv], buf, gsem).start()

            def wait_g(idxv, bufs, gsem):
                for t_ref, buf in zip(t_refs, bufs):
                    pltpu.make_async_copy(t_ref.at[idxv], buf, gsem).wait()

            def put(c, bufs, osem):
                off = base + c * chunk
                for o_ref, buf in zip(o_refs, bufs):
                    pltpu.make_async_copy(buf, o_ref.at[pl.ds(off, chunk)],
                                          osem).start()

            def wait_p(bufs, osem):
                for o_ref, buf in zip(o_refs, bufs):
                    pltpu.make_async_copy(buf, o_ref.at[pl.ds(base, chunk)],
                                          osem).wait()

            issue(0, idx_a, bufs_a, gs_a)

            def loop_body(j, carry):
                c0 = 2 * j
                c1 = c0 + 1

                @pl.when(j > 0)
                def _():
                    wait_p(bufs_b, os_b)
                issue(c1, idx_b, bufs_b, gs_b)
                wait_g(idx_a, bufs_a, gs_a)
                put(c0, bufs_a, os_a)

                @pl.when(c0 + 2 < nchunk)
                def _():
                    wait_p(bufs_a, os_a)
                    issue(c0 + 2, idx_a, bufs_a, gs_a)
                wait_g(idx_b, bufs_b, gs_b)
                put(c1, bufs_b, os_b)
                return carry

            lax.fori_loop(0, nchunk // 2, loop_body, 0)
            wait_p(bufs_a, os_a)
            wait_p(bufs_b, os_b)

    f = pl.kernel(
        body,
        out_type=tuple(jax.ShapeDtypeStruct((G,) + s, d)
                       for s, d in zip(row_shapes, row_dtypes)),
        mesh=mesh,
        scratch_types=scratch,
    )
    out = f(*tables, idx)
    out = list(out) if isinstance(out, (tuple, list)) else [out]
    return [o if od is None else o[:, :od]
            for o, od in zip(out, orig_dims)]


def _attn(p, xyz_pad, feats, q, kg, vg, xg):
    B, N, dp = feats.shape
    TM = min(N, 128)
    wd1 = _pad_rows(p["fc_delta1"]["W"], XP)
    scale = 1.0 / np.sqrt(DM)

    def body(x_ref, f_ref, q_ref, k_ref, v_ref, xg_ref,
             wd1_ref, bd1_ref, wd2_ref, bd2_ref, wg1_ref, bg1_ref,
             wg2_ref, bg2_ref, w2_ref, b2_ref, o_ref):
        xi = x_ref[0]
        xg3 = xg_ref[0]
        delta = (xi.reshape(TM, 1, XP) - xg3).reshape(TM * KNB, XP)
        pos1 = jnp.maximum(_dot(delta, wd1_ref[...]) + bd1_ref[...], 0.0)
        pos = _dot(pos1, wd2_ref[...]) + bd2_ref[...]
        q3 = q_ref[0].reshape(TM, 1, DM)
        g0 = (q3 - k_ref[0].astype(F32)).reshape(TM * KNB, DM) + pos
        g1 = jnp.maximum(_dot(g0, wg1_ref[...]) + bg1_ref[...], 0.0)
        logits = (_dot(g1, wg2_ref[...]) + bg2_ref[...]) * scale
        a3 = logits.reshape(TM, KNB, DM)
        m = jnp.max(a3, axis=1, keepdims=True)
        e = jnp.exp(a3 - m)
        w3 = e / jnp.sum(e, axis=1, keepdims=True)
        vpe = v_ref[0].astype(F32) + pos.reshape(TM, KNB, DM)
        out = jnp.sum(w3 * vpe, axis=1)
        o_ref[0] = _dot(out, w2_ref[...]) + b2_ref[...] + f_ref[0]

    wb = lambda shape: pl.BlockSpec(shape, lambda b, m: (0, 0))
    return pl.pallas_call(
        body,
        grid=(B, N // TM),
        in_specs=[
            pl.BlockSpec((1, TM, XP), lambda b, m: (b, m, 0)),
            pl.BlockSpec((1, TM, dp), lambda b, m: (b, m, 0)),
            pl.BlockSpec((1, TM, DM), lambda b, m: (b, m, 0)),
            pl.BlockSpec((1, TM, KNB, DM), lambda b, m: (b, m, 0, 0)),
            pl.BlockSpec((1, TM, KNB, DM), lambda b, m: (b, m, 0, 0)),
            pl.BlockSpec((1, TM, KNB, XP), lambda b, m: (b, m, 0, 0)),
            wb(wd1.shape), wb((1, DM)),
            wb((DM, DM)), wb((1, DM)),
            wb((DM, DM)), wb((1, DM)),
            wb((DM, DM)), wb((1, DM)),
            wb((DM, dp)), wb((1, dp)),
        ],
        out_specs=pl.BlockSpec((1, TM, dp), lambda b, m: (b, m, 0)),
        out_shape=jax.ShapeDtypeStruct((B, N, dp), F32),
    )(xyz_pad, feats, q, kg, vg, xg,
      wd1, p["fc_delta1"]["b"].reshape(1, DM),
      p["fc_delta2"]["W"], p["fc_delta2"]["b"].reshape(1, DM),
      p["fc_gamma1"]["W"], p["fc_gamma1"]["b"].reshape(1, DM),
      p["fc_gamma2"]["W"], p["fc_gamma2"]["b"].reshape(1, DM),
      p["fc2"]["W"], p["fc2"]["b"].reshape(1, dp))


def _fps(xyzT, npnt):
    B, _, N = xyzT.shape

    def body(x_ref, o_ref):
        xv = x_ref[...]
        iota = lax.broadcasted_iota(jnp.int32, (B, 1, N), 2)
        bofs = lax.broadcasted_iota(jnp.int32, (B, 1, 1), 0) * N

        def step(i, carry):
            dist, f = carry
            o_ref[:, pl.ds(i, 1), :] = f + bofs
            mask = (iota == f).astype(F32)
            cm = jnp.sum(xv * mask, axis=2, keepdims=True)
            d = jnp.sum((xv - cm) ** 2, axis=1, keepdims=True)
            dist = jnp.minimum(dist, d)
            mv = jnp.max(dist, axis=2, keepdims=True)
            f2 = jnp.min(jnp.where(dist == mv, iota, N), axis=2,
                         keepdims=True)
            return dist, f2

        lax.fori_loop(0, npnt, step,
                      (jnp.full((B, 1, N), 1e10, F32),
                       jnp.zeros((B, 1, 1), jnp.int32)))

    return pl.pallas_call(
        body,
        grid=(1,),
        in_specs=[pl.BlockSpec((B, 4, N), lambda i: (0, 0, 0))],
        out_specs=pl.BlockSpec((B, npnt, 1), lambda i: (0, 0, 0)),
        out_shape=jax.ShapeDtypeStruct((B, npnt, 1), jnp.int32),
    )(xyzT)


def _acc_stats(h, s_ref, ss_ref):
    s = jnp.sum(h, axis=0, keepdims=True)
    ss = jnp.sum(h * h, axis=0, keepdims=True)

    @pl.when(pl.program_id(0) == 0)
    def _():
        s_ref[...] = s
        ss_ref[...] = ss

    @pl.when(pl.program_id(0) > 0)
    def _():
        s_ref[...] += s
        ss_ref[...] += ss


def _bn_from_stats(h, s_ref, ss_ref, g_ref, be_ref, n):
    mean = s_ref[...] * (1.0 / n)
    var = ss_ref[...] * (1.0 / n) - mean * mean
    hn = (h - mean) / jnp.sqrt(var + 1e-5)
    return jnp.maximum(g_ref[...] * hn + be_ref[...], 0.0)


def _sa(layers, gsa, nx):
    R = gsa.shape[0]
    M = R // KNB
    C = layers[0]["W"].shape[1]
    w1x = _pad_rows(layers[0]["W"][:3], XP)
    w1p = _pad_rows(jnp.pad(layers[0]["W"][3:], ((XP, 0), (0, 0))), 128)
    TR = min(R, 4096)
    grid = (R // TR,)
    row = lambda width: pl.BlockSpec((TR, width), lambda i: (i, 0))
    cst = lambda shape: pl.BlockSpec(shape, lambda i: (0,) * len(shape))
    stat_spec = [cst((1, C)), cst((1, C))]
    stat_shape = [jax.ShapeDtypeStruct((1, C), F32)] * 2

    def k1(g_ref, nx_ref, w1x_ref, w1p_ref, b1_ref,
           h_ref, s_ref, ss_ref):
        g = g_ref[...]
        gx3 = g[:, :XP].reshape(TR // KNB, KNB, XP)
        nx3 = nx_ref[...].reshape(TR // KNB, 1, XP)
        gnorm = (gx3 - nx3).reshape(TR, XP)
        h = (_dot(gnorm, w1x_ref[...]) + _dot(g, w1p_ref[...])
             + b1_ref[...])
        h_ref[...] = h
        _acc_stats(h, s_ref, ss_ref)

    h1, s1, ss1 = pl.pallas_call(
        k1, grid=grid,
        in_specs=[row(128),
                  pl.BlockSpec((TR // KNB, XP), lambda i: (i, 0)),
                  cst(w1x.shape), cst(w1p.shape), cst((1, C))],
        out_specs=[row(C)] + stat_spec,
        out_shape=[jax.ShapeDtypeStruct((R, C), F32)] + stat_shape,
    )(gsa, nx, w1x, w1p, layers[0]["b"].reshape(1, C))

    def k2(h_ref, s1_ref, ss1_ref, g1_ref, be1_ref, w2_ref, b2_ref,
           h2_ref, s_ref, ss_ref):
        hn = _bn_from_stats(h_ref[...], s1_ref, ss1_ref, g1_ref, be1_ref, R)
        h2 = _dot(hn, w2_ref[...]) + b2_ref[...]
        h2_ref[...] = h2
        _acc_stats(h2, s_ref, ss_ref)

    h2, s2, ss2 = pl.pallas_call(
        k2, grid=grid,
        in_specs=[row(C), cst((1, C)), cst((1, C)), cst((1, C)), cst((1, C)),
                  cst((C, C)), cst((1, C))],
        out_specs=[row(C)] + stat_spec,
        out_shape=[jax.ShapeDtypeStruct((R, C), F32)] + stat_shape,
    )(h1, s1, ss1, layers[0]["gamma"].reshape(1, C),
      layers[0]["beta"].reshape(1, C), layers[1]["W"],
      layers[1]["b"].reshape(1, C))

    def k3(h2_ref, s2_ref, ss2_ref, g2_ref, be2_ref, o_ref):
        hn = _bn_from_stats(h2_ref[...], s2_ref, ss2_ref, g2_ref, be2_ref, R)
        o_ref[...] = jnp.max(hn.reshape(TR // KNB, KNB, C), axis=1)

    return pl.pallas_call(
        k3, grid=grid,
        in_specs=[row(C), cst((1, C)), cst((1, C)), cst((1, C)),
                  cst((1, C))],
        out_specs=pl.BlockSpec((TR // KNB, C), lambda i: (i, 0)),
        out_shape=jax.ShapeDtypeStruct((M, C), F32),
    )(h2, s2, ss2, layers[1]["gamma"].reshape(1, C),
      layers[1]["beta"].reshape(1, C))


def _tblock(p, xyz_pad, feats):
    B, N, dp = feats.shape
    q, kp, vp = _proj(feats, p)
    knn_g = _knn(xyz_pad, xyz_pad)
    gflat = knn_g.reshape(B * N * KNB)
    kg, vg, xg = _sc_gather(
        [kp.reshape(B * N, DM), vp.reshape(B * N, DM),
         xyz_pad.reshape(B * N, XP)], gflat)
    return _attn(p, xyz_pad, feats, q,
                 kg.reshape(B, N, KNB, DM), vg.reshape(B, N, KNB, DM),
                 xg.reshape(B, N, KNB, XP))


def kernel(x, params):
    B, N, _ = x.shape
    xyz = x[..., :3]
    xyz_pad = _pad_cols(xyz, XP)
    feats = _mlp0(_pad_cols(x, XP), params["fc1a"], params["fc1b"])
    points = _tblock(params["t1"], xyz_pad, feats)
    xyz_and_feats = [(xyz, points)]
    cur_xyzp, cur_points = xyz_pad, points
    cur_n = N
    for i in range(4):
        npnt = N // 2 ** (i + 1)
        fps_g = _fps(jnp.swapaxes(cur_xyzp[..., :4], 1, 2), npnt)
        fps_flat = fps_g.reshape(B * npnt)
        (new_xyzp_flat,) = _sc_gather(
            [cur_xyzp.reshape(B * cur_n, XP)], fps_flat)
        new_xyzp = new_xyzp_flat.reshape(B, npnt, XP)
        knn_g = _knn(new_xyzp, cur_xyzp)
        gflat = knn_g.reshape(B * npnt * KNB)
        sa_tab = _pad_cols(jnp.concatenate([cur_xyzp, cur_points], axis=-1),
                           128)
        (gsa,) = _sc_gather([sa_tab.reshape(B * cur_n, 128)], gflat)
        new_points = _sa(params["td"][i], gsa,
                         new_xyzp.reshape(B * npnt, XP))
        new_points = new_points.reshape(B, npnt, -1)
        cur_points = _tblock(params["tf"][i], new_xyzp, new_points)
        cur_xyzp = new_xyzp
        cur_n = npnt
        xyz_and_feats.append((new_xyzp[..., :3], cur_points))
    return cur_points, xyz_and_feats
